# baseline (speedup 1.0000x reference)
import functools

import jax
import jax.numpy as jnp
from jax import lax
from jax.experimental import pallas as pl
from jax.experimental.pallas import tpu as pltpu
from jax.experimental.pallas import tpu_sc as plsc

_N = 10000
_F = 256
_NS = 16
_NC = 2
_NT = _NC * _NS
_K = 128
_EPT = 10240
_EROWS = _EPT // _K
_CAP_ROWS = 88
_NBKT = 8
_BSZ = 1256
_BOUNDS = (0, 1256, 2512, 3768, 5024, 6280, 7536, 8792, 10000)
_ACC = 2 * _BSZ


def _cdiv(a, b):
    return (a + b - 1) // b


_RB = 1000
_GRID = _N // _RB


def _tc0_body(f_ref, wl_ref, bl_ref, w1_ref, p_ref, h_ref):
    p = jnp.dot(f_ref[...], wl_ref[...], preferred_element_type=jnp.float32)
    p = p + bl_ref[...]
    p_ref[...] = p
    x = jax.nn.sigmoid(p)
    h_ref[...] = jnp.dot(x, w1_ref[...], preferred_element_type=jnp.float32)


def _tc0(features, W_lin, b_lin, W1):
    return pl.pallas_call(
        _tc0_body,
        grid=(_GRID,),
        in_specs=[
            pl.BlockSpec((_RB, 128), lambda i: (i, 0)),
            pl.BlockSpec((128, _F), lambda i: (0, 0)),
            pl.BlockSpec((1, _F), lambda i: (0, 0)),
            pl.BlockSpec((_F, _F), lambda i: (0, 0)),
        ],
        out_specs=[
            pl.BlockSpec((_RB, _F), lambda i: (i, 0)),
            pl.BlockSpec((_RB, _F), lambda i: (i, 0)),
        ],
        out_shape=[
            jax.ShapeDtypeStruct((_N, _F), jnp.float32),
            jax.ShapeDtypeStruct((_N, _F), jnp.float32),
        ],
    )(features, W_lin, b_lin.reshape(1, _F), W1)


def _tc_mid_body(s_ref, b_ref, w_ref, g_ref, h_ref):
    g = s_ref[...] + b_ref[...]
    g_ref[...] = g
    x = jnp.maximum(g, 0.0)
    h_ref[...] = jnp.dot(x, w_ref[...], preferred_element_type=jnp.float32)


def _tc_mid(s, b, W):
    return pl.pallas_call(
        _tc_mid_body,
        grid=(_GRID,),
        in_specs=[
            pl.BlockSpec((_RB, _F), lambda i: (i, 0)),
            pl.BlockSpec((1, _F), lambda i: (0, 0)),
            pl.BlockSpec((_F, _F), lambda i: (0, 0)),
        ],
        out_specs=[
            pl.BlockSpec((_RB, _F), lambda i: (i, 0)),
            pl.BlockSpec((_RB, _F), lambda i: (i, 0)),
        ],
        out_shape=[
            jax.ShapeDtypeStruct((_N, _F), jnp.float32),
            jax.ShapeDtypeStruct((_N, _F), jnp.float32),
        ],
    )(s, b.reshape(1, _F), W)


def _tc_bias_body(s_ref, b_ref, g_ref):
    g_ref[...] = s_ref[...] + b_ref[...]


def _tc_bias(s, b):
    return pl.pallas_call(
        _tc_bias_body,
        grid=(_GRID,),
        in_specs=[
            pl.BlockSpec((_RB, _F), lambda i: (i, 0)),
            pl.BlockSpec((1, _F), lambda i: (0, 0)),
        ],
        out_specs=pl.BlockSpec((_RB, _F), lambda i: (i, 0)),
        out_shape=jax.ShapeDtypeStruct((_N, _F), jnp.float32),
    )(s, b.reshape(1, _F))


def _part_body(src_hbm, dst_hbm, w_hbm, srcp_hbm, dstp_hbm, wp_hbm, cnt_hbm,
               srcv, dstv, wv, stg_s, stg_d, stg_w, blk_s, blk_d, blk_w, cntv):
    c = lax.axis_index("c")
    s = lax.axis_index("s")
    t = c * _NS + s

    pltpu.sync_copy(src_hbm.at[t], srcv)
    pltpu.sync_copy(dst_hbm.at[t], dstv)
    pltpu.sync_copy(w_hbm.at[t], wv)

    nsteps = _EPT // 16
    zero_i = jnp.zeros((16,), jnp.int32)
    zero_f = jnp.zeros((16,), jnp.float32)

    def step(i, carry):
        fills = carry[0:_NBKT]
        rowis = carry[_NBKT:2 * _NBKT]
        blkss = carry[2 * _NBKT:3 * _NBKT]
        cntrs = carry[3 * _NBKT:4 * _NBKT]
        pad = i >= nsteps
        iota16 = lax.broadcasted_iota(jnp.int32, (16,), 0)
        r = jnp.minimum(i // 8, _EROWS - 1)
        cg = lax.rem(i, 8)
        sl = pl.ds(cg * 16, 16)
        vsrc = jnp.where(pad, zero_i, srcv[r, sl])
        vdst = jnp.where(pad, zero_i, dstv[r, sl])
        vw = jnp.where(pad, zero_f, wv[r, sl])

        new = []
        for b in range(_NBKT):
            lo, hi = _BOUNDS[b], _BOUNDS[b + 1]
            m = jnp.logical_or(
                jnp.logical_and(vdst >= lo, vdst < hi), pad)
            spread = iota16 * 75 + lax.rem(i - nsteps, 8) * 9
            vloc = jnp.where(pad, spread, vdst - lo)
            fill, rowi, blks, cntr = fills[b], rowis[b], blkss[b], cntrs[b]
            mi = jnp.where(m, jnp.full((16,), 1, jnp.int32), zero_i)
            pos = plsc.cumsum(mi)
            idx = b * 160 + fill + pos - 1
            plsc.store_scatter(stg_s, [idx], vsrc, mask=m)
            plsc.store_scatter(stg_d, [idx], vloc, mask=m)
            plsc.store_scatter(stg_w, [idx], vw, mask=m)
            pc = pos[15]
            fill2 = fill + pc
            flush = fill2 >= _K
            blk_full = jnp.logical_and(flush, rowi == 7)

            @pl.when(flush)
            def _(b=b, rowi=rowi, blks=blks, blk_full=blk_full):
                for kk in range(8):
                    ssl = pl.ds(b * 160 + kk * 16, 16)
                    dsl = pl.ds(kk * 16, 16)
                    blk_s[b, rowi, dsl] = stg_s[ssl]
                    blk_d[b, rowi, dsl] = stg_d[ssl]
                    blk_w[b, rowi, dsl] = stg_w[ssl]
                vs = stg_s[pl.ds(b * 160 + _K, 16)]
                vd = stg_d[pl.ds(b * 160 + _K, 16)]
                vw2 = stg_w[pl.ds(b * 160 + _K, 16)]
                stg_s[pl.ds(b * 160, 16)] = vs
                stg_d[pl.ds(b * 160, 16)] = vd
                stg_w[pl.ds(b * 160, 16)] = vw2

                @pl.when(blk_full)
                def _(b=b, blks=blks):
                    off = pl.multiple_of(blks * 8, 8)
                    pltpu.sync_copy(blk_s.at[b], srcp_hbm.at[b, t, pl.ds(off, 8)])
                    pltpu.sync_copy(blk_d.at[b], dstp_hbm.at[b, t, pl.ds(off, 8)])
                    pltpu.sync_copy(blk_w.at[b], wp_hbm.at[b, t, pl.ds(off, 8)])

            new.append((
                jnp.where(flush, fill2 - _K, fill2),
                jnp.where(blk_full, 0, jnp.where(flush, rowi + 1, rowi)),
                jnp.where(blk_full, blks + 1, blks),
                cntr + jnp.where(pad, 0, pc),
            ))
        return (tuple(x[0] for x in new) + tuple(x[1] for x in new)
                + tuple(x[2] for x in new) + tuple(x[3] for x in new))

    init = (jnp.int32(0),) * (4 * _NBKT)
    fin = lax.fori_loop(0, nsteps + 8, step, init)
    rowis = fin[_NBKT:2 * _NBKT]
    blkss = fin[2 * _NBKT:3 * _NBKT]
    cntrs = fin[3 * _NBKT:4 * _NBKT]

    for b in range(_NBKT):
        off = pl.multiple_of(blkss[b] * 8, 8)

        @pl.when(rowis[b] > 0)
        def _(b=b, off=off):
            pltpu.sync_copy(blk_s.at[b], srcp_hbm.at[b, t, pl.ds(off, 8)])
            pltpu.sync_copy(blk_d.at[b], dstp_hbm.at[b, t, pl.ds(off, 8)])
            pltpu.sync_copy(blk_w.at[b], wp_hbm.at[b, t, pl.ds(off, 8)])

    iota = lax.broadcasted_iota(jnp.int32, (16,), 0)
    cv = jnp.zeros((16,), jnp.int32)
    for b in range(_NBKT):
        cv = jnp.where(iota == b, cntrs[b], cv)
    cntv[0, pl.ds(0, 16)] = cv
    pltpu.sync_copy(cntv, cnt_hbm.at[t])


_partition = functools.partial(
    pl.kernel,
    out_type=[
        jax.ShapeDtypeStruct((_NBKT, _NT, _CAP_ROWS, _K), jnp.int32),
        jax.ShapeDtypeStruct((_NBKT, _NT, _CAP_ROWS, _K), jnp.int32),
        jax.ShapeDtypeStruct((_NBKT, _NT, _CAP_ROWS, _K), jnp.float32),
        jax.ShapeDtypeStruct((_NT, 8, _K), jnp.int32),
    ],
    mesh=plsc.VectorSubcoreMesh(core_axis_name="c", subcore_axis_name="s"),
    compiler_params=pltpu.CompilerParams(needs_layout_passes=False),
    scratch_types=[
        pltpu.VMEM((_EROWS, _K), jnp.int32),
        pltpu.VMEM((_EROWS, _K), jnp.int32),
        pltpu.VMEM((_EROWS, _K), jnp.float32),
        pltpu.VMEM((_NBKT * 160,), jnp.int32),
        pltpu.VMEM((_NBKT * 160,), jnp.int32),
        pltpu.VMEM((_NBKT * 160,), jnp.float32),
        pltpu.VMEM((_NBKT, 8, _K), jnp.int32),
        pltpu.VMEM((_NBKT, 8, _K), jnp.int32),
        pltpu.VMEM((_NBKT, 8, _K), jnp.float32),
        pltpu.VMEM((8, _K), jnp.int32),
    ],
)(_part_body)


def _spmm_body(h_hbm, srcp, dstp, wp, cntp, out_hbm,
               acc, src_t, dst_t, dst2_t, w_t, rows_v, sc_buf, cnt_v, sem):
    c = lax.axis_index("c")
    s = lax.axis_index("s")

    zero = jnp.zeros((16,), jnp.float32)

    def _stage(par, b, t, goff):
        pltpu.sync_copy(srcp.at[b, t, pl.ds(goff, 8)], src_t.at[par])
        pltpu.sync_copy(dstp.at[b, t, pl.ds(goff, 8)], dst_t.at[par])
        pltpu.sync_copy(wp.at[b, t, pl.ds(goff, 8)], w_t.at[par])
        for r8 in range(8):
            for kk in range(8):
                sl = pl.ds(kk * 16, 16)
                d2 = dst_t[par, r8, sl] * 2
                dst2_t[par, 2 * r8, sl] = d2
                dst2_t[par, 2 * r8 + 1, sl] = d2 + 1

    def one_pass(q, pcarry):
        b = 2 * q + c
        nblk = jnp.where(b == _NBKT - 1,
                         2 * (_BOUNDS[8] - _BOUNDS[7]) // 16, _ACC // 16)

        for r in range(16):
            for v in range(8):
                sc_buf[r, pl.ds(v * 16, 16)] = zero
        for k in range(_cdiv(_ACC // 16, _NS)):
            blk = s + _NS * k

            @pl.when(blk < _ACC // 16)
            def _(blk=blk):
                off = pl.multiple_of(blk * 16, 8)
                pltpu.sync_copy(sc_buf.at[pl.ds(0, 16)], acc.at[pl.ds(off, 16)])
        plsc.subcore_barrier()

        for rgn in range(2):
            t = 2 * s + rgn
            pltpu.sync_copy(cntp.at[t], cnt_v)
            cvec = cnt_v[0, pl.ds(0, 16)]
            ce = jnp.where(q == 0, cvec[0],
                           jnp.where(q == 1, cvec[2],
                                     jnp.where(q == 2, cvec[4], cvec[6])))
            co = jnp.where(q == 0, cvec[1],
                           jnp.where(q == 1, cvec[3],
                                     jnp.where(q == 2, cvec[5], cvec[7])))
            cnt = jnp.where(c == 0, ce, co)
            nch = lax.div(cnt + _K - 1, _K)

            @pl.when(nch > 0)
            def _(t=t, b=b):
                _stage(0, b, t, 0)

            def chunk(j, carry, t=t, b=b, nch=nch):
                gi = j // 8
                ji = j - gi * 8
                p = lax.rem(j, 2)
                gp = lax.rem(gi, 2)

                @pl.when(jnp.logical_and(ji == 0, (gi + 1) * 8 < nch))
                def _():
                    goff = pl.multiple_of((gi + 1) * 8, 8)
                    _stage(1 - gp, b, t, goff)

                pltpu.async_copy(h_hbm.at[src_t.at[gp, ji]],
                                 rows_v.at[p], sem).wait()

                rp = rows_v.at[p]

                def group16(g, c2):
                    wvec = w_t[gp, ji, pl.ds(g * 16, 16)]
                    for el in range(16):
                        wsv = zero + wvec[el]
                        e = g * 16 + el
                        for v in range(8):
                            sl = pl.ds(v * 16, 16)
                            sc_buf[e, sl] = rp[e, sl] * wsv
                            sc_buf[_K + e, sl] = rp[e, pl.ds(128 + v * 16, 16)] * wsv
                    return c2

                lax.fori_loop(0, _K // 16, group16, 0, unroll=2)
                pltpu.sync_copy(sc_buf.at[pl.ds(0, _K)],
                                acc.at[dst2_t.at[gp, 2 * ji]], add=True)
                pltpu.sync_copy(sc_buf.at[pl.ds(_K, _K)],
                                acc.at[dst2_t.at[gp, 2 * ji + 1]], add=True)
                return carry

            lax.fori_loop(0, nch, chunk, 0)
        plsc.subcore_barrier()

        base = b * _ACC
        for k in range(_cdiv(_ACC // 16, _NS)):
            blk = s + _NS * k

            @pl.when(blk < nblk)
            def _(blk=blk):
                off = pl.multiple_of(blk * 16, 8)
                dof = pl.multiple_of(base + blk * 16, 8)
                pltpu.sync_copy(acc.at[pl.ds(off, 16)],
                                out_hbm.at[pl.ds(dof, 16)])

        return pcarry

    lax.fori_loop(0, 4, one_pass, 0)


_spmm = functools.partial(
    pl.kernel,
    out_type=jax.ShapeDtypeStruct((2 * _N, 128), jnp.float32),
    mesh=plsc.VectorSubcoreMesh(core_axis_name="c", subcore_axis_name="s"),
    scratch_types=[
        pltpu.VMEM_SHARED((_ACC, 128), jnp.float32),
        pltpu.VMEM((2, 8, _K), jnp.int32),
        pltpu.VMEM((2, 8, _K), jnp.int32),
        pltpu.VMEM((2, 16, _K), jnp.int32),
        pltpu.VMEM((2, 8, _K), jnp.float32),
        pltpu.VMEM((2, _K, _F), jnp.float32),
        pltpu.VMEM((2 * _K, 128), jnp.float32),
        pltpu.VMEM((8, _K), jnp.int32),
        pltpu.SemaphoreType.DMA,
    ],
)(_spmm_body)


def kernel(features, edge_index, edge_weight, W_lin, b_lin, W1, b1, W2, b2):
    e = edge_index.shape[1]
    e_pad = _NT * _EPT
    src = edge_index[0]
    dst = edge_index[1]
    pad = e_pad - e
    if pad:
        zi = jnp.zeros((pad,), jnp.int32)
        src = jnp.concatenate([src, zi])
        dst = jnp.concatenate([dst, (jnp.arange(pad, dtype=jnp.int32) * 13)
                               % _N])
        w = jnp.concatenate([edge_weight, jnp.zeros((pad,), jnp.float32)])
    else:
        w = edge_weight
    src_r = src.reshape(_NT, _EROWS, _K)
    dst_r = dst.reshape(_NT, _EROWS, _K)
    w_r = w.reshape(_NT, _EROWS, _K)

    srcp, dstp, wp, cntp = _partition(src_r, dst_r, w_r)
    p, h1 = _tc0(features, W_lin, b_lin, W1)
    s1 = _spmm(h1, srcp, dstp, wp, cntp).reshape(_N, _F)
    g1, h2 = _tc_mid(s1, b1, W2)
    s2 = _spmm(h2, srcp, dstp, wp, cntp).reshape(_N, _F)
    g2, h3 = _tc_mid(s2, b2, W2)
    s3 = _spmm(h3, srcp, dstp, wp, cntp).reshape(_N, _F)
    g3 = _tc_bias(s3, b2)
    return jnp.concatenate([p, g1, g2, g3], axis=1)


# speedup vs baseline: 2.1839x; 2.1839x over previous
import functools

import jax
import jax.numpy as jnp
from jax import lax
from jax.experimental import pallas as pl
from jax.experimental.pallas import tpu as pltpu
from jax.experimental.pallas import tpu_sc as plsc

_N = 10000
_D = 128
_NS = 16
_NC = 2
_K = 128
_ROW_BLK = 80
_NBLK = _N // _ROW_BLK


def _cdiv(a, b):
    return (a + b - 1) // b


_RB = 1000
_GRID = _N // _RB


def _tc0_body(f_ref, wl_ref, bl_ref, w1_ref, p_ref, h_ref):
    p = jnp.dot(f_ref[...], wl_ref[...], preferred_element_type=jnp.float32)
    p = p + bl_ref[...]
    p_ref[...] = p
    x = jax.nn.sigmoid(p)
    h = jnp.dot(x, w1_ref[...], preferred_element_type=jnp.float32)
    h_ref[0] = h[:, :_D]
    h_ref[1] = h[:, _D:]


def _tc0(features, W_lin, b_lin, W1):
    return pl.pallas_call(
        _tc0_body,
        grid=(_GRID,),
        in_specs=[
            pl.BlockSpec((_RB, 128), lambda i: (i, 0)),
            pl.BlockSpec((128, 256), lambda i: (0, 0)),
            pl.BlockSpec((1, 256), lambda i: (0, 0)),
            pl.BlockSpec((256, 256), lambda i: (0, 0)),
        ],
        out_specs=[
            pl.BlockSpec((_RB, 256), lambda i: (i, 0)),
            pl.BlockSpec((2, _RB, _D), lambda i: (0, i, 0)),
        ],
        out_shape=[
            jax.ShapeDtypeStruct((_N, 256), jnp.float32),
            jax.ShapeDtypeStruct((2, _N, _D), jnp.float32),
        ],
    )(features, W_lin, b_lin.reshape(1, 256), W1)


def _tc_mid_body(s_ref, b_ref, w_ref, g_ref, h_ref):
    g = jnp.concatenate([s_ref[0], s_ref[1]], axis=1) + b_ref[...]
    g_ref[...] = g
    x = jnp.maximum(g, 0.0)
    h = jnp.dot(x, w_ref[...], preferred_element_type=jnp.float32)
    h_ref[0] = h[:, :_D]
    h_ref[1] = h[:, _D:]


def _tc_mid(s_split, b, W):
    return pl.pallas_call(
        _tc_mid_body,
        grid=(_GRID,),
        in_specs=[
            pl.BlockSpec((2, _RB, _D), lambda i: (0, i, 0)),
            pl.BlockSpec((1, 256), lambda i: (0, 0)),
            pl.BlockSpec((256, 256), lambda i: (0, 0)),
        ],
        out_specs=[
            pl.BlockSpec((_RB, 256), lambda i: (i, 0)),
            pl.BlockSpec((2, _RB, _D), lambda i: (0, i, 0)),
        ],
        out_shape=[
            jax.ShapeDtypeStruct((_N, 256), jnp.float32),
            jax.ShapeDtypeStruct((2, _N, _D), jnp.float32),
        ],
    )(s_split, b.reshape(1, 256), W)


def _tc_bias_body(s_ref, b_ref, g_ref):
    g_ref[...] = jnp.concatenate([s_ref[0], s_ref[1]], axis=1) + b_ref[...]


def _tc_bias(s_split, b):
    return pl.pallas_call(
        _tc_bias_body,
        grid=(_GRID,),
        in_specs=[
            pl.BlockSpec((2, _RB, _D), lambda i: (0, i, 0)),
            pl.BlockSpec((1, 256), lambda i: (0, 0)),
        ],
        out_specs=pl.BlockSpec((_RB, 256), lambda i: (i, 0)),
        out_shape=jax.ShapeDtypeStruct((_N, 256), jnp.float32),
    )(s_split, b.reshape(1, 256))


def _spmm_body(h_hbm, src_hbm, dst_hbm, w_hbm, out_hbm,
               acc, src_t, dst_t, w_t, rows_v, sem, nchunk):
    c = lax.axis_index("c")
    s = lax.axis_index("s")

    zero = jnp.zeros((16,), jnp.float32)

    def zrow(r, carry):
        for v in range(8):
            rows_v[r, pl.ds(v * 16, 16)] = zero
        return carry

    lax.fori_loop(0, _K, zrow, 0)
    for b in range(_cdiv(_NBLK, _NS)):
        blk = s + _NS * b

        @pl.when(blk < _NBLK)
        def _(blk=blk):
            pltpu.sync_copy(
                rows_v.at[pl.ds(0, _ROW_BLK)],
                acc.at[pl.ds(blk * _ROW_BLK, _ROW_BLK)],
            )
    plsc.subcore_barrier()

    def grp(jo, carry):
        pltpu.sync_copy(src_hbm.at[c, s, pl.ds(jo * 8, 8)], src_t)
        pltpu.sync_copy(dst_hbm.at[s, pl.ds(jo * 8, 8)], dst_t)
        pltpu.sync_copy(w_hbm.at[s, pl.ds(jo * 8, 8)], w_t)

        def one(ji, c1):
            pltpu.async_copy(h_hbm.at[src_t.at[ji]], rows_v, sem).wait()

            def group(g, c2):
                wvec = w_t[ji, pl.ds(g * 16, 16)]
                for el in range(16):
                    ws = wvec[el]
                    e = g * 16 + el
                    for v in range(8):
                        sl = pl.ds(v * 16, 16)
                        rows_v[e, sl] = rows_v[e, sl] * ws
                return c2

            lax.fori_loop(0, _K // 16, group, 0, unroll=2)
            pltpu.sync_copy(rows_v, acc.at[dst_t.at[ji]], add=True)
            return c1

        lax.fori_loop(0, 8, one, 0)
        return carry

    lax.fori_loop(0, nchunk // 8, grp, 0)
    plsc.subcore_barrier()

    for b in range(_cdiv(_NBLK, _NS)):
        blk = s + _NS * b

        @pl.when(blk < _NBLK)
        def _(blk=blk):
            pltpu.sync_copy(
                acc.at[pl.ds(blk * _ROW_BLK, _ROW_BLK)],
                out_hbm.at[c, pl.ds(blk * _ROW_BLK, _ROW_BLK)],
            )


def _make_spmm(nchunk):
    return functools.partial(
        pl.kernel,
        out_type=jax.ShapeDtypeStruct((_NC, _N, _D), jnp.float32),
        mesh=plsc.VectorSubcoreMesh(core_axis_name="c", subcore_axis_name="s"),
        scratch_types=[
            pltpu.VMEM_SHARED((_N, _D), jnp.float32),
            pltpu.VMEM((8, _K), jnp.int32),
            pltpu.VMEM((8, _K), jnp.int32),
            pltpu.VMEM((8, _K), jnp.float32),
            pltpu.VMEM((_K, _D), jnp.float32),
            pltpu.SemaphoreType.DMA,
        ],
    )(functools.partial(_spmm_body, nchunk=nchunk))


def kernel(features, edge_index, edge_weight, W_lin, b_lin, W1, b1, W2, b2):
    n = features.shape[0]
    e = edge_index.shape[1]
    nchunk = 8 * _cdiv(e, _NS * _K * 8)
    e_pad = _NS * nchunk * _K

    src = edge_index[0]
    dst = edge_index[1]
    pad = e_pad - e
    if pad:
        zi = jnp.zeros((pad,), jnp.int32)
        src = jnp.concatenate([src, zi])
        dst = jnp.concatenate([dst, (jnp.arange(pad, dtype=jnp.int32) * 13)
                               % n])
        w = jnp.concatenate([edge_weight, jnp.zeros((pad,), jnp.float32)])
    else:
        w = edge_weight
    src2 = jnp.stack([src, src + n]).reshape(_NC, _NS, nchunk, _K)
    dst_r = dst.reshape(_NS, nchunk, _K)
    w_r = w.reshape(_NS, nchunk, _K)

    spmm = _make_spmm(nchunk)

    p, h1 = _tc0(features, W_lin, b_lin, W1)
    s1 = spmm(h1.reshape(_NC * n, _D), src2, dst_r, w_r)
    g1, h2 = _tc_mid(s1, b1, W2)
    s2 = spmm(h2.reshape(_NC * n, _D), src2, dst_r, w_r)
    g2, h3 = _tc_mid(s2, b2, W2)
    s3 = spmm(h3.reshape(_NC * n, _D), src2, dst_r, w_r)
    g3 = _tc_bias(s3, b2)
    return jnp.concatenate([p, g1, g2, g3], axis=1)


# speedup vs baseline: 2.2722x; 1.0404x over previous
import functools

import jax
import jax.numpy as jnp
from jax import lax
from jax.experimental import pallas as pl
from jax.experimental.pallas import tpu as pltpu
from jax.experimental.pallas import tpu_sc as plsc

_N = 10000
_D = 128
_NS = 16
_NC = 2
_K = 128
_ROW_BLK = 80
_NBLK = _N // _ROW_BLK


def _cdiv(a, b):
    return (a + b - 1) // b


_RB = 1000
_GRID = _N // _RB


def _tc0_body(f_ref, wl_ref, bl_ref, w1_ref, p_ref, h_ref):
    p = jnp.dot(f_ref[...], wl_ref[...], preferred_element_type=jnp.float32)
    p = p + bl_ref[...]
    p_ref[...] = p
    x = jax.nn.sigmoid(p)
    h = jnp.dot(x, w1_ref[...], preferred_element_type=jnp.float32)
    h_ref[0] = h[:, :_D]
    h_ref[1] = h[:, _D:]


def _tc0(features, W_lin, b_lin, W1):
    return pl.pallas_call(
        _tc0_body,
        grid=(_GRID,),
        in_specs=[
            pl.BlockSpec((_RB, 128), lambda i: (i, 0)),
            pl.BlockSpec((128, 256), lambda i: (0, 0)),
            pl.BlockSpec((1, 256), lambda i: (0, 0)),
            pl.BlockSpec((256, 256), lambda i: (0, 0)),
        ],
        out_specs=[
            pl.BlockSpec((_RB, 256), lambda i: (i, 0)),
            pl.BlockSpec((2, _RB, _D), lambda i: (0, i, 0)),
        ],
        out_shape=[
            jax.ShapeDtypeStruct((_N, 256), jnp.float32),
            jax.ShapeDtypeStruct((2, _N, _D), jnp.float32),
        ],
    )(features, W_lin, b_lin.reshape(1, 256), W1)


def _tc_mid_body(s_ref, b_ref, w_ref, g_ref, h_ref):
    g = jnp.concatenate([s_ref[0], s_ref[1]], axis=1) + b_ref[...]
    g_ref[...] = g
    x = jnp.maximum(g, 0.0)
    h = jnp.dot(x, w_ref[...], preferred_element_type=jnp.float32)
    h_ref[0] = h[:, :_D]
    h_ref[1] = h[:, _D:]


def _tc_mid(s_split, b, W):
    return pl.pallas_call(
        _tc_mid_body,
        grid=(_GRID,),
        in_specs=[
            pl.BlockSpec((2, _RB, _D), lambda i: (0, i, 0)),
            pl.BlockSpec((1, 256), lambda i: (0, 0)),
            pl.BlockSpec((256, 256), lambda i: (0, 0)),
        ],
        out_specs=[
            pl.BlockSpec((_RB, 256), lambda i: (i, 0)),
            pl.BlockSpec((2, _RB, _D), lambda i: (0, i, 0)),
        ],
        out_shape=[
            jax.ShapeDtypeStruct((_N, 256), jnp.float32),
            jax.ShapeDtypeStruct((2, _N, _D), jnp.float32),
        ],
    )(s_split, b.reshape(1, 256), W)


def _tc_bias_body(s_ref, b_ref, g_ref):
    g_ref[...] = jnp.concatenate([s_ref[0], s_ref[1]], axis=1) + b_ref[...]


def _tc_bias(s_split, b):
    return pl.pallas_call(
        _tc_bias_body,
        grid=(_GRID,),
        in_specs=[
            pl.BlockSpec((2, _RB, _D), lambda i: (0, i, 0)),
            pl.BlockSpec((1, 256), lambda i: (0, 0)),
        ],
        out_specs=pl.BlockSpec((_RB, 256), lambda i: (i, 0)),
        out_shape=jax.ShapeDtypeStruct((_N, 256), jnp.float32),
    )(s_split, b.reshape(1, 256))


def _spmm_body(h_hbm, src_hbm, dst_hbm, w_hbm, out_hbm,
               acc, src_t, dst_t, w_t, rows_v, rows_w, sem, sem2, nchunk):
    c = lax.axis_index("c")
    s = lax.axis_index("s")

    zero = jnp.zeros((16,), jnp.float32)

    def zrow(r, carry):
        for v in range(8):
            rows_v[r, pl.ds(v * 16, 16)] = zero
        return carry

    lax.fori_loop(0, _K, zrow, 0)
    for b in range(_cdiv(_NBLK, _NS)):
        blk = s + _NS * b

        @pl.when(blk < _NBLK)
        def _(blk=blk):
            pltpu.sync_copy(
                rows_v.at[pl.ds(0, _ROW_BLK)],
                acc.at[pl.ds(blk * _ROW_BLK, _ROW_BLK)],
            )
    plsc.subcore_barrier()

    def grp(jo, carry):
        pltpu.sync_copy(src_hbm.at[c, s, pl.ds(jo * 8, 8)], src_t)
        pltpu.sync_copy(dst_hbm.at[s, pl.ds(jo * 8, 8)], dst_t)
        pltpu.sync_copy(w_hbm.at[s, pl.ds(jo * 8, 8)], w_t)

        def pair(jj, c1):
            ja = 2 * jj
            jb = 2 * jj + 1
            da = pltpu.async_copy(h_hbm.at[src_t.at[ja]], rows_v, sem)
            db = pltpu.async_copy(h_hbm.at[src_t.at[jb]], rows_w, sem2)
            da.wait()

            def group_a(g, c2):
                wvec = w_t[ja, pl.ds(g * 16, 16)]
                for el in range(16):
                    ws = wvec[el]
                    e = g * 16 + el
                    for v in range(8):
                        sl = pl.ds(v * 16, 16)
                        rows_v[e, sl] = rows_v[e, sl] * ws
                return c2

            lax.fori_loop(0, _K // 16, group_a, 0, unroll=2)
            pltpu.sync_copy(rows_v, acc.at[dst_t.at[ja]], add=True)
            db.wait()

            def group_b(g, c2):
                wvec = w_t[jb, pl.ds(g * 16, 16)]
                for el in range(16):
                    ws = wvec[el]
                    e = g * 16 + el
                    for v in range(8):
                        sl = pl.ds(v * 16, 16)
                        rows_w[e, sl] = rows_w[e, sl] * ws
                return c2

            lax.fori_loop(0, _K // 16, group_b, 0, unroll=2)
            pltpu.sync_copy(rows_w, acc.at[dst_t.at[jb]], add=True)
            return c1

        lax.fori_loop(0, 4, pair, 0)
        return carry

    lax.fori_loop(0, nchunk // 8, grp, 0)
    plsc.subcore_barrier()

    for b in range(_cdiv(_NBLK, _NS)):
        blk = s + _NS * b

        @pl.when(blk < _NBLK)
        def _(blk=blk):
            pltpu.sync_copy(
                acc.at[pl.ds(blk * _ROW_BLK, _ROW_BLK)],
                out_hbm.at[c, pl.ds(blk * _ROW_BLK, _ROW_BLK)],
            )


def _make_spmm(nchunk):
    return functools.partial(
        pl.kernel,
        out_type=jax.ShapeDtypeStruct((_NC, _N, _D), jnp.float32),
        mesh=plsc.VectorSubcoreMesh(core_axis_name="c", subcore_axis_name="s"),
        scratch_types=[
            pltpu.VMEM_SHARED((_N, _D), jnp.float32),
            pltpu.VMEM((8, _K), jnp.int32),
            pltpu.VMEM((8, _K), jnp.int32),
            pltpu.VMEM((8, _K), jnp.float32),
            pltpu.VMEM((_K, _D), jnp.float32),
            pltpu.VMEM((_K, _D), jnp.float32),
            pltpu.SemaphoreType.DMA,
            pltpu.SemaphoreType.DMA,
        ],
    )(functools.partial(_spmm_body, nchunk=nchunk))


def kernel(features, edge_index, edge_weight, W_lin, b_lin, W1, b1, W2, b2):
    n = features.shape[0]
    e = edge_index.shape[1]
    nchunk = 8 * _cdiv(e, _NS * _K * 8)
    e_pad = _NS * nchunk * _K

    src = edge_index[0]
    dst = edge_index[1]
    pad = e_pad - e
    if pad:
        zi = jnp.zeros((pad,), jnp.int32)
        src = jnp.concatenate([src, zi])
        dst = jnp.concatenate([dst, (jnp.arange(pad, dtype=jnp.int32) * 13)
                               % n])
        w = jnp.concatenate([edge_weight, jnp.zeros((pad,), jnp.float32)])
    else:
        w = edge_weight
    src2 = jnp.stack([src, src + n]).reshape(_NC, _NS, nchunk, _K)
    dst_r = dst.reshape(_NS, nchunk, _K)
    w_r = w.reshape(_NS, nchunk, _K)

    spmm = _make_spmm(nchunk)

    p, h1 = _tc0(features, W_lin, b_lin, W1)
    s1 = spmm(h1.reshape(_NC * n, _D), src2, dst_r, w_r)
    g1, h2 = _tc_mid(s1, b1, W2)
    s2 = spmm(h2.reshape(_NC * n, _D), src2, dst_r, w_r)
    g2, h3 = _tc_mid(s2, b2, W2)
    s3 = spmm(h3.reshape(_NC * n, _D), src2, dst_r, w_r)
    g3 = _tc_bias(s3, b2)
    return jnp.concatenate([p, g1, g2, g3], axis=1)


# speedup vs baseline: 2.2786x; 1.0028x over previous
import functools

import jax
import jax.numpy as jnp
from jax import lax
from jax.experimental import pallas as pl
from jax.experimental.pallas import tpu as pltpu
from jax.experimental.pallas import tpu_sc as plsc

_N = 10000
_D = 128
_NS = 16
_NC = 2
_K = 128
_ROW_BLK = 80
_NBLK = _N // _ROW_BLK


def _cdiv(a, b):
    return (a + b - 1) // b


_RB = 1000
_GRID = _N // _RB


def _tc0_body(f_ref, wl_ref, bl_ref, w1_ref, p_ref, h_ref):
    p = jnp.dot(f_ref[...], wl_ref[...], preferred_element_type=jnp.float32)
    p = p + bl_ref[...]
    p_ref[...] = p
    x = jax.nn.sigmoid(p)
    h = jnp.dot(x, w1_ref[...], preferred_element_type=jnp.float32)
    h_ref[0] = h[:, :_D]
    h_ref[1] = h[:, _D:]


def _tc0(features, W_lin, b_lin, W1):
    return pl.pallas_call(
        _tc0_body,
        grid=(_GRID,),
        in_specs=[
            pl.BlockSpec((_RB, 128), lambda i: (i, 0)),
            pl.BlockSpec((128, 256), lambda i: (0, 0)),
            pl.BlockSpec((1, 256), lambda i: (0, 0)),
            pl.BlockSpec((256, 256), lambda i: (0, 0)),
        ],
        out_specs=[
            pl.BlockSpec((_RB, 256), lambda i: (i, 0)),
            pl.BlockSpec((2, _RB, _D), lambda i: (0, i, 0)),
        ],
        out_shape=[
            jax.ShapeDtypeStruct((_N, 256), jnp.float32),
            jax.ShapeDtypeStruct((2, _N, _D), jnp.float32),
        ],
    )(features, W_lin, b_lin.reshape(1, 256), W1)


def _tc_mid_body(s_ref, b_ref, w_ref, g_ref, h_ref):
    g = jnp.concatenate([s_ref[0], s_ref[1]], axis=1) + b_ref[...]
    g_ref[...] = g
    x = jnp.maximum(g, 0.0)
    h = jnp.dot(x, w_ref[...], preferred_element_type=jnp.float32)
    h_ref[0] = h[:, :_D]
    h_ref[1] = h[:, _D:]


def _tc_mid(s_split, b, W):
    return pl.pallas_call(
        _tc_mid_body,
        grid=(_GRID,),
        in_specs=[
            pl.BlockSpec((2, _RB, _D), lambda i: (0, i, 0)),
            pl.BlockSpec((1, 256), lambda i: (0, 0)),
            pl.BlockSpec((256, 256), lambda i: (0, 0)),
        ],
        out_specs=[
            pl.BlockSpec((_RB, 256), lambda i: (i, 0)),
            pl.BlockSpec((2, _RB, _D), lambda i: (0, i, 0)),
        ],
        out_shape=[
            jax.ShapeDtypeStruct((_N, 256), jnp.float32),
            jax.ShapeDtypeStruct((2, _N, _D), jnp.float32),
        ],
    )(s_split, b.reshape(1, 256), W)


def _tc_bias_body(s_ref, b_ref, g_ref):
    g_ref[...] = jnp.concatenate([s_ref[0], s_ref[1]], axis=1) + b_ref[...]


def _tc_bias(s_split, b):
    return pl.pallas_call(
        _tc_bias_body,
        grid=(_GRID,),
        in_specs=[
            pl.BlockSpec((2, _RB, _D), lambda i: (0, i, 0)),
            pl.BlockSpec((1, 256), lambda i: (0, 0)),
        ],
        out_specs=pl.BlockSpec((_RB, 256), lambda i: (i, 0)),
        out_shape=jax.ShapeDtypeStruct((_N, 256), jnp.float32),
    )(s_split, b.reshape(1, 256))


def _spmm_body(h_hbm, src_hbm, dst_hbm, w_hbm, out_hbm,
               acc, src_t, dst_t, w_t, rows_v, rows_w, sem, sem2, nchunk):
    c = lax.axis_index("c")
    s = lax.axis_index("s")

    zero = jnp.zeros((16,), jnp.float32)

    def zrow(r, carry):
        for v in range(8):
            rows_v[r, pl.ds(v * 16, 16)] = zero
        return carry

    lax.fori_loop(0, _K, zrow, 0)
    for b in range(_cdiv(_NBLK, _NS)):
        blk = s + _NS * b

        @pl.when(blk < _NBLK)
        def _(blk=blk):
            pltpu.sync_copy(
                rows_v.at[pl.ds(0, _ROW_BLK)],
                acc.at[pl.ds(blk * _ROW_BLK, _ROW_BLK)],
            )
    plsc.subcore_barrier()

    def grp(jo, carry):
        pltpu.sync_copy(src_hbm.at[c, s, pl.ds(jo * 8, 8)], src_t)
        pltpu.sync_copy(dst_hbm.at[s, pl.ds(jo * 8, 8)], dst_t)
        pltpu.sync_copy(w_hbm.at[s, pl.ds(jo * 8, 8)], w_t)

        def pair(jj, c1):
            ja = 2 * jj
            jb = 2 * jj + 1
            da = pltpu.async_copy(h_hbm.at[src_t.at[ja]], rows_v, sem)
            db = pltpu.async_copy(h_hbm.at[src_t.at[jb]], rows_w, sem2)
            da.wait()

            def group_a(g, c2):
                wvec = w_t[ja, pl.ds(g * 16, 16)]
                for el in range(16):
                    ws = wvec[el]
                    e = g * 16 + el
                    for v in range(8):
                        sl = pl.ds(v * 16, 16)
                        rows_v[e, sl] = rows_v[e, sl] * ws
                return c2

            lax.fori_loop(0, _K // 16, group_a, 0, unroll=4)
            pltpu.sync_copy(rows_v, acc.at[dst_t.at[ja]], add=True)
            db.wait()

            def group_b(g, c2):
                wvec = w_t[jb, pl.ds(g * 16, 16)]
                for el in range(16):
                    ws = wvec[el]
                    e = g * 16 + el
                    for v in range(8):
                        sl = pl.ds(v * 16, 16)
                        rows_w[e, sl] = rows_w[e, sl] * ws
                return c2

            lax.fori_loop(0, _K // 16, group_b, 0, unroll=4)
            pltpu.sync_copy(rows_w, acc.at[dst_t.at[jb]], add=True)
            return c1

        lax.fori_loop(0, 4, pair, 0)
        return carry

    lax.fori_loop(0, nchunk // 8, grp, 0)
    plsc.subcore_barrier()

    for b in range(_cdiv(_NBLK, _NS)):
        blk = s + _NS * b

        @pl.when(blk < _NBLK)
        def _(blk=blk):
            pltpu.sync_copy(
                acc.at[pl.ds(blk * _ROW_BLK, _ROW_BLK)],
                out_hbm.at[c, pl.ds(blk * _ROW_BLK, _ROW_BLK)],
            )


def _make_spmm(nchunk):
    return functools.partial(
        pl.kernel,
        out_type=jax.ShapeDtypeStruct((_NC, _N, _D), jnp.float32),
        mesh=plsc.VectorSubcoreMesh(core_axis_name="c", subcore_axis_name="s"),
        scratch_types=[
            pltpu.VMEM_SHARED((_N, _D), jnp.float32),
            pltpu.VMEM((8, _K), jnp.int32),
            pltpu.VMEM((8, _K), jnp.int32),
            pltpu.VMEM((8, _K), jnp.float32),
            pltpu.VMEM((_K, _D), jnp.float32),
            pltpu.VMEM((_K, _D), jnp.float32),
            pltpu.SemaphoreType.DMA,
            pltpu.SemaphoreType.DMA,
        ],
    )(functools.partial(_spmm_body, nchunk=nchunk))


def kernel(features, edge_index, edge_weight, W_lin, b_lin, W1, b1, W2, b2):
    n = features.shape[0]
    e = edge_index.shape[1]
    nchunk = 8 * _cdiv(e, _NS * _K * 8)
    e_pad = _NS * nchunk * _K

    src = edge_index[0]
    dst = edge_index[1]
    pad = e_pad - e
    if pad:
        zi = jnp.zeros((pad,), jnp.int32)
        src = jnp.concatenate([src, zi])
        dst = jnp.concatenate([dst, (jnp.arange(pad, dtype=jnp.int32) * 13)
                               % n])
        w = jnp.concatenate([edge_weight, jnp.zeros((pad,), jnp.float32)])
    else:
        w = edge_weight
    src2 = jnp.stack([src, src + n]).reshape(_NC, _NS, nchunk, _K)
    dst_r = dst.reshape(_NS, nchunk, _K)
    w_r = w.reshape(_NS, nchunk, _K)

    spmm = _make_spmm(nchunk)

    p, h1 = _tc0(features, W_lin, b_lin, W1)
    s1 = spmm(h1.reshape(_NC * n, _D), src2, dst_r, w_r)
    g1, h2 = _tc_mid(s1, b1, W2)
    s2 = spmm(h2.reshape(_NC * n, _D), src2, dst_r, w_r)
    g2, h3 = _tc_mid(s2, b2, W2)
    s3 = spmm(h3.reshape(_NC * n, _D), src2, dst_r, w_r)
    g3 = _tc_bias(s3, b2)
    return jnp.concatenate([p, g1, g2, g3], axis=1)


# speedup vs baseline: 2.3179x; 1.0172x over previous
import functools

import jax
import jax.numpy as jnp
from jax import lax
from jax.experimental import pallas as pl
from jax.experimental.pallas import tpu as pltpu
from jax.experimental.pallas import tpu_sc as plsc

_N = 10000
_D = 128
_NS = 16
_NC = 2
_K = 128
_ROW_BLK = 80
_NBLK = _N // _ROW_BLK


def _cdiv(a, b):
    return (a + b - 1) // b


_RB = 1000
_GRID = _N // _RB


def _tc0_body(f_ref, wl_ref, bl_ref, w1_ref, p_ref, h_ref):
    p = jnp.dot(f_ref[...], wl_ref[...], preferred_element_type=jnp.float32)
    p = p + bl_ref[...]
    p_ref[...] = p
    x = jax.nn.sigmoid(p)
    h = jnp.dot(x, w1_ref[...], preferred_element_type=jnp.float32)
    h_ref[0] = h[:, :_D]
    h_ref[1] = h[:, _D:]


def _tc0(features, W_lin, b_lin, W1):
    return pl.pallas_call(
        _tc0_body,
        grid=(_GRID,),
        in_specs=[
            pl.BlockSpec((_RB, 128), lambda i: (i, 0)),
            pl.BlockSpec((128, 256), lambda i: (0, 0)),
            pl.BlockSpec((1, 256), lambda i: (0, 0)),
            pl.BlockSpec((256, 256), lambda i: (0, 0)),
        ],
        out_specs=[
            pl.BlockSpec((_RB, 256), lambda i: (i, 0)),
            pl.BlockSpec((2, _RB, _D), lambda i: (0, i, 0)),
        ],
        out_shape=[
            jax.ShapeDtypeStruct((_N, 256), jnp.float32),
            jax.ShapeDtypeStruct((2, _N, _D), jnp.float32),
        ],
    )(features, W_lin, b_lin.reshape(1, 256), W1)


def _tc_mid_body(s_ref, b_ref, w_ref, g_ref, h_ref):
    g = jnp.concatenate([s_ref[0], s_ref[1]], axis=1) + b_ref[...]
    g_ref[...] = g
    x = jnp.maximum(g, 0.0)
    h = jnp.dot(x, w_ref[...], preferred_element_type=jnp.float32)
    h_ref[0] = h[:, :_D]
    h_ref[1] = h[:, _D:]


def _tc_mid(s_split, b, W):
    return pl.pallas_call(
        _tc_mid_body,
        grid=(_GRID,),
        in_specs=[
            pl.BlockSpec((2, _RB, _D), lambda i: (0, i, 0)),
            pl.BlockSpec((1, 256), lambda i: (0, 0)),
            pl.BlockSpec((256, 256), lambda i: (0, 0)),
        ],
        out_specs=[
            pl.BlockSpec((_RB, 256), lambda i: (i, 0)),
            pl.BlockSpec((2, _RB, _D), lambda i: (0, i, 0)),
        ],
        out_shape=[
            jax.ShapeDtypeStruct((_N, 256), jnp.float32),
            jax.ShapeDtypeStruct((2, _N, _D), jnp.float32),
        ],
    )(s_split, b.reshape(1, 256), W)


def _tc_bias_body(s_ref, b_ref, g_ref):
    g_ref[...] = jnp.concatenate([s_ref[0], s_ref[1]], axis=1) + b_ref[...]


def _tc_bias(s_split, b):
    return pl.pallas_call(
        _tc_bias_body,
        grid=(_GRID,),
        in_specs=[
            pl.BlockSpec((2, _RB, _D), lambda i: (0, i, 0)),
            pl.BlockSpec((1, 256), lambda i: (0, 0)),
        ],
        out_specs=pl.BlockSpec((_RB, 256), lambda i: (i, 0)),
        out_shape=jax.ShapeDtypeStruct((_N, 256), jnp.float32),
    )(s_split, b.reshape(1, 256))


def _spmm_body(h_hbm, src_hbm, dst_hbm, w_hbm, out_hbm,
               acc, src_t, dst_t, w_t, rows_v, rows_w, sem, sem2, nchunk):
    c = lax.axis_index("c")
    s = lax.axis_index("s")

    zero = jnp.zeros((16,), jnp.float32)

    def zrow(r, carry):
        for v in range(8):
            rows_v[r, pl.ds(v * 16, 16)] = zero
        return carry

    lax.fori_loop(0, _K, zrow, 0)
    for b in range(_cdiv(_NBLK, _NS)):
        blk = s + _NS * b

        @pl.when(blk < _NBLK)
        def _(blk=blk):
            pltpu.sync_copy(
                rows_v.at[pl.ds(0, _ROW_BLK)],
                acc.at[pl.ds(blk * _ROW_BLK, _ROW_BLK)],
            )
    plsc.subcore_barrier()

    def grp(jo, carry):
        pltpu.sync_copy(src_hbm.at[c, s, pl.ds(jo * 16, 16)], src_t)
        pltpu.sync_copy(dst_hbm.at[s, pl.ds(jo * 16, 16)], dst_t)
        pltpu.sync_copy(w_hbm.at[s, pl.ds(jo * 16, 16)], w_t)

        def pair(jj, c1):
            ja = 2 * jj
            jb = 2 * jj + 1
            da = pltpu.async_copy(h_hbm.at[src_t.at[ja]], rows_v, sem)
            db = pltpu.async_copy(h_hbm.at[src_t.at[jb]], rows_w, sem2)
            da.wait()

            def group_a(g, c2):
                wvec = w_t[ja, pl.ds(g * 16, 16)]
                for el in range(16):
                    ws = wvec[el]
                    e = g * 16 + el
                    for v in range(8):
                        sl = pl.ds(v * 16, 16)
                        rows_v[e, sl] = rows_v[e, sl] * ws
                return c2

            lax.fori_loop(0, _K // 16, group_a, 0, unroll=4)
            pltpu.sync_copy(rows_v, acc.at[dst_t.at[ja]], add=True)
            db.wait()

            def group_b(g, c2):
                wvec = w_t[jb, pl.ds(g * 16, 16)]
                for el in range(16):
                    ws = wvec[el]
                    e = g * 16 + el
                    for v in range(8):
                        sl = pl.ds(v * 16, 16)
                        rows_w[e, sl] = rows_w[e, sl] * ws
                return c2

            lax.fori_loop(0, _K // 16, group_b, 0, unroll=4)
            pltpu.sync_copy(rows_w, acc.at[dst_t.at[jb]], add=True)
            return c1

        lax.fori_loop(0, 8, pair, 0)
        return carry

    lax.fori_loop(0, nchunk // 16, grp, 0)
    plsc.subcore_barrier()

    for b in range(_cdiv(_NBLK, _NS)):
        blk = s + _NS * b

        @pl.when(blk < _NBLK)
        def _(blk=blk):
            pltpu.sync_copy(
                acc.at[pl.ds(blk * _ROW_BLK, _ROW_BLK)],
                out_hbm.at[c, pl.ds(blk * _ROW_BLK, _ROW_BLK)],
            )


def _make_spmm(nchunk):
    return functools.partial(
        pl.kernel,
        out_type=jax.ShapeDtypeStruct((_NC, _N, _D), jnp.float32),
        mesh=plsc.VectorSubcoreMesh(core_axis_name="c", subcore_axis_name="s"),
        scratch_types=[
            pltpu.VMEM_SHARED((_N, _D), jnp.float32),
            pltpu.VMEM((16, _K), jnp.int32),
            pltpu.VMEM((16, _K), jnp.int32),
            pltpu.VMEM((16, _K), jnp.float32),
            pltpu.VMEM((_K, _D), jnp.float32),
            pltpu.VMEM((_K, _D), jnp.float32),
            pltpu.SemaphoreType.DMA,
            pltpu.SemaphoreType.DMA,
        ],
    )(functools.partial(_spmm_body, nchunk=nchunk))


def kernel(features, edge_index, edge_weight, W_lin, b_lin, W1, b1, W2, b2):
    n = features.shape[0]
    e = edge_index.shape[1]
    nchunk = 16 * _cdiv(e, _NS * _K * 16)
    e_pad = _NS * nchunk * _K

    src = edge_index[0]
    dst = edge_index[1]
    pad = e_pad - e
    if pad:
        zi = jnp.zeros((pad,), jnp.int32)
        src = jnp.concatenate([src, zi])
        dst = jnp.concatenate([dst, (jnp.arange(pad, dtype=jnp.int32) * 13)
                               % n])
        w = jnp.concatenate([edge_weight, jnp.zeros((pad,), jnp.float32)])
    else:
        w = edge_weight
    src2 = jnp.stack([src, src + n]).reshape(_NC, _NS, nchunk, _K)
    dst_r = dst.reshape(_NS, nchunk, _K)
    w_r = w.reshape(_NS, nchunk, _K)

    spmm = _make_spmm(nchunk)

    p, h1 = _tc0(features, W_lin, b_lin, W1)
    s1 = spmm(h1.reshape(_NC * n, _D), src2, dst_r, w_r)
    g1, h2 = _tc_mid(s1, b1, W2)
    s2 = spmm(h2.reshape(_NC * n, _D), src2, dst_r, w_r)
    g2, h3 = _tc_mid(s2, b2, W2)
    s3 = spmm(h3.reshape(_NC * n, _D), src2, dst_r, w_r)
    g3 = _tc_bias(s3, b2)
    return jnp.concatenate([p, g1, g2, g3], axis=1)
